# baseline (speedup 1.0000x reference)
import jax
import jax.numpy as jnp
from jax import lax
from jax.experimental import pallas as pl
from jax.experimental.pallas import tpu as pltpu
from jax.experimental.pallas import tpu_sc as plsc

N_NODES = 100000
H = 128
C = 10
CP = 16
NW = 32
CHUNK = 3200
NP = NW * CHUNK
TCB = 12800
MT = TCB // 8
SPW = TCB // CHUNK
NG = 256
GX = 64
NGRP = MT // 16


def _nodewise_body(init_ref, fin_ref, wg_ref, wt_ref, bg_ref, bt_ref, out_ref):
    i = pl.program_id(0)
    a = jnp.dot(init_ref[...], wg_ref[0:H, :],
                preferred_element_type=jnp.float32)
    b = jnp.dot(fin_ref[...], wg_ref[H:2 * H, :],
                preferred_element_type=jnp.float32)
    t = jnp.dot(fin_ref[...], wt_ref[...],
                preferred_element_type=jnp.float32)
    gate = jax.nn.sigmoid(a + b + bg_ref[...])
    nodewise = jnp.concatenate(
        [gate * (t + bt_ref[...]), jnp.zeros((TCB, CP - C), jnp.float32)],
        axis=1)
    row = i * TCB + lax.broadcasted_iota(jnp.int32, (TCB, CP), 0)
    nodewise = jnp.where(row < N_NODES, nodewise, 0.0)
    out_ref[...] = jnp.concatenate(
        [nodewise[c * MT:(c + 1) * MT] for c in range(8)], axis=1)


RPW = NP // 8 // NW
RUN = RPW
QG = RUN // 16


def _segsum_body(nw_hbm, ids_hbm, out_hbm, rows_v, ids_v, acc_v,
                 sem_rows, sem_ids):
    cid = lax.axis_index("c")
    sid = lax.axis_index("s")
    wid = sid * 2 + cid
    cp_rows = pltpu.async_copy(
        nw_hbm.at[pl.ds(wid * RPW, RPW), :], rows_v, sem_rows)
    blk = wid // SPW
    lw = wid % SPW
    starts = []
    cps = []
    for c in range(8):
        start_c = blk * TCB + c * MT + lw * RUN
        base_c = jnp.minimum(start_c, N_NODES - RUN)
        cps.append(pltpu.async_copy(ids_hbm.at[pl.ds(base_c, RUN)],
                                    ids_v.at[pl.ds(c * RUN, RUN)], sem_ids))
        starts.append(start_c - base_c + c * RUN)

    zeros16 = jnp.zeros((CP,), jnp.float32)
    iota16 = lax.iota(jnp.int32, CP)

    def acc_idx(g):
        return [jnp.full((CP,), g & 31, jnp.int32),
                ((g >> 5) & 7) * 16 + iota16]

    def zero_body(r, carry):
        for c in range(8):
            acc_v[r, pl.ds(c * 16, 16)] = zeros16
        return carry

    lax.fori_loop(0, NG // 8, zero_body, 0)
    for cp in cps:
        cp.wait()
    cp_rows.wait()

    def make_group_body(c, idbase):
        cb = c * 16

        def group_body(q, carry):
            ids_grp = ids_v[pl.ds(idbase + q * 16, 16)]
            first = ids_grp[0]
            last = ids_grp[15]
            p0 = q * 16

            def fast(_):
                s = rows_v[p0, pl.ds(cb, 16)]
                for r in range(1, 16):
                    s = s + rows_v[p0 + r, pl.ds(cb, 16)]
                plsc.addupdate_scatter(acc_v, acc_idx(first), s)
                return 0

            def slow(_):
                for r in range(16):
                    g = ids_grp[r]
                    plsc.addupdate_scatter(
                        acc_v, acc_idx(g), rows_v[p0 + r, pl.ds(cb, 16)])
                return 0

            lax.cond(first == last, fast, slow, 0)
            return carry
        return group_body

    for c in range(8):
        lax.fori_loop(0, QG, make_group_body(c, starts[c]), 0, unroll=5)
    pltpu.sync_copy(acc_v, out_hbm.at[wid])


def _finalize_body(part_ref, aux_ref, gm_ref, bt_ref,
                   w1_ref, b1_ref, w2_ref, b2_ref, out_ref):
    grp = jnp.sum(part_ref[...], axis=0)
    gr = jnp.concatenate(
        [grp[:, c * 16:(c + 1) * 16] for c in range(8)], axis=0)[:, :C]
    m = jnp.mean(gr, axis=0, keepdims=True)
    v = jnp.mean((gr - m) ** 2, axis=0, keepdims=True)
    ngr = ((gr - m) * lax.rsqrt(v + 1e-5) * gm_ref[:, :C]
           + bt_ref[:, :C])
    ax = aux_ref[...]
    ma = jnp.mean(ax, axis=0, keepdims=True)
    va = jnp.mean((ax - ma) ** 2, axis=0, keepdims=True)
    nax = ((ax - ma) * lax.rsqrt(va + 1e-5) * gm_ref[:, C:]
           + bt_ref[:, C:])
    h = jnp.dot(ngr, w1_ref[0:C, :], preferred_element_type=jnp.float32)
    h = h + jnp.dot(nax, w1_ref[C:, :], preferred_element_type=jnp.float32)
    h = jnp.maximum(h + b1_ref[...], 0.0)
    out_ref[...] = (
        jnp.dot(h, w2_ref[...], preferred_element_type=jnp.float32)
        + b2_ref[...])


def kernel(initial_node_states, final_node_states, aux_variables, num_graphs,
           graph_nodes_list, Wg, bg, Wt, bt, gamma, beta, W1, b1, W2, b2):
    f32 = jnp.float32
    del num_graphs
    ids = jnp.asarray(graph_nodes_list, jnp.int32)

    nodewise = pl.pallas_call(
        _nodewise_body,
        grid=(NP // TCB,),
        in_specs=[
            pl.BlockSpec((TCB, H), lambda i: (i, 0)),
            pl.BlockSpec((TCB, H), lambda i: (i, 0)),
            pl.BlockSpec((2 * H, C), lambda i: (0, 0)),
            pl.BlockSpec((H, C), lambda i: (0, 0)),
            pl.BlockSpec((1, C), lambda i: (0, 0)),
            pl.BlockSpec((1, C), lambda i: (0, 0)),
        ],
        out_specs=pl.BlockSpec((MT, 128), lambda i: (i, 0)),
        out_shape=jax.ShapeDtypeStruct((NP // 8, 128), f32),
    )(initial_node_states, final_node_states, Wg,
      Wt, bg.reshape(1, C), bt.reshape(1, C))

    mesh = plsc.VectorSubcoreMesh(core_axis_name="c", subcore_axis_name="s")
    partials = pl.kernel(
        _segsum_body,
        out_type=jax.ShapeDtypeStruct((NW, NG // 8, 128), f32),
        mesh=mesh,
        scratch_types=[
            pltpu.VMEM((RPW, 128), f32),
            pltpu.VMEM((CHUNK + 2432, ), jnp.int32),
            pltpu.VMEM((NG // 8, 128), f32),
            pltpu.SemaphoreType.DMA,
            pltpu.SemaphoreType.DMA,
        ],
        compiler_params=pltpu.CompilerParams(
            needs_layout_passes=False, use_tc_tiling_on_sc=False),
    )(nodewise, ids)

    logits = pl.pallas_call(
        _finalize_body,
        out_shape=jax.ShapeDtypeStruct((NG, C), f32),
    )(partials, aux_variables, gamma.reshape(1, C + 2),
      beta.reshape(1, C + 2), W1, b1.reshape(1, GX), W2, b2.reshape(1, C))
    return logits


# speedup vs baseline: 1.0748x; 1.0748x over previous
import jax
import jax.numpy as jnp
from jax import lax
from jax.experimental import pallas as pl
from jax.experimental.pallas import tpu as pltpu
from jax.experimental.pallas import tpu_sc as plsc

N_NODES = 100000
H = 128
C = 10
CP = 16
NW = 32
CHUNK = 3200
NP = NW * CHUNK
TCB = 12800
MT = TCB // 8
SPW = TCB // CHUNK
NG = 256
GX = 64
NGRP = MT // 16


def _nodewise_body(init_ref, fin_ref, wg_ref, wt_ref, bg_ref, bt_ref, out_ref):
    i = pl.program_id(0)
    a = jnp.dot(init_ref[...], wg_ref[0:H, :],
                preferred_element_type=jnp.float32)
    b = jnp.dot(fin_ref[...], wg_ref[H:2 * H, :],
                preferred_element_type=jnp.float32)
    t = jnp.dot(fin_ref[...], wt_ref[...],
                preferred_element_type=jnp.float32)
    gate = jax.nn.sigmoid(a + b + bg_ref[...])
    nodewise = jnp.concatenate(
        [gate * (t + bt_ref[...]), jnp.zeros((TCB, CP - C), jnp.float32)],
        axis=1)
    row = i * TCB + lax.broadcasted_iota(jnp.int32, (TCB, CP), 0)
    nodewise = jnp.where(row < N_NODES, nodewise, 0.0)
    out_ref[...] = jnp.concatenate(
        [nodewise[c * MT:(c + 1) * MT] for c in range(8)], axis=1)


RPW = NP // 8 // NW
RUN = RPW
QG = RUN // 16


def _segsum_body(nw_hbm, ids_hbm, out_hbm, rows_v, ids_v, acc_v,
                 sem_rows, sem_ids):
    cid = lax.axis_index("c")
    sid = lax.axis_index("s")
    wid = sid * 2 + cid
    cp_rows = pltpu.async_copy(
        nw_hbm.at[pl.ds(wid * RPW, RPW), :], rows_v, sem_rows)
    blk = wid // SPW
    lw = wid % SPW
    starts = []
    cps = []
    for c in range(8):
        start_c = blk * TCB + c * MT + lw * RUN
        base_c = jnp.minimum(start_c, N_NODES - RUN)
        cps.append(pltpu.async_copy(ids_hbm.at[pl.ds(base_c, RUN)],
                                    ids_v.at[pl.ds(c * RUN, RUN)], sem_ids))
        starts.append(start_c - base_c + c * RUN)

    zeros16 = jnp.zeros((CP,), jnp.float32)
    iota16 = lax.iota(jnp.int32, CP)

    def acc_idx(g):
        return [jnp.full((CP,), g & 31, jnp.int32),
                ((g >> 5) & 7) * 16 + iota16]

    def zero_body(r, carry):
        for c in range(8):
            acc_v[r, pl.ds(c * 16, 16)] = zeros16
        return carry

    lax.fori_loop(0, NG // 8, zero_body, 0)
    for cp in cps:
        cp.wait()
    cp_rows.wait()

    def make_group_body(c, idbase):
        cb = c * 16

        def group_body(q, carry):
            ids_grp = ids_v[pl.ds(idbase + q * 16, 16)]
            first = ids_grp[0]
            last = ids_grp[15]
            p0 = q * 16

            def fast(_):
                s = rows_v[p0, pl.ds(cb, 16)]
                for r in range(1, 16):
                    s = s + rows_v[p0 + r, pl.ds(cb, 16)]
                plsc.addupdate_scatter(acc_v, acc_idx(first), s)
                return 0

            def slow(_):
                for r in range(16):
                    g = ids_grp[r]
                    plsc.addupdate_scatter(
                        acc_v, acc_idx(g), rows_v[p0 + r, pl.ds(cb, 16)])
                return 0

            lax.cond(first == last, fast, slow, 0)
            return carry
        return group_body

    for c in range(8):
        lax.fori_loop(0, QG, make_group_body(c, starts[c]), 0)
    pltpu.sync_copy(acc_v, out_hbm.at[wid])


def _finalize_body(part_ref, aux_ref, gm_ref, bt_ref,
                   w1_ref, b1_ref, w2_ref, b2_ref, out_ref):
    grp = jnp.sum(part_ref[...], axis=0)
    gr = jnp.concatenate(
        [grp[:, c * 16:(c + 1) * 16] for c in range(8)], axis=0)[:, :C]
    m = jnp.mean(gr, axis=0, keepdims=True)
    v = jnp.mean((gr - m) ** 2, axis=0, keepdims=True)
    ngr = ((gr - m) * lax.rsqrt(v + 1e-5) * gm_ref[:, :C]
           + bt_ref[:, :C])
    ax = aux_ref[...]
    ma = jnp.mean(ax, axis=0, keepdims=True)
    va = jnp.mean((ax - ma) ** 2, axis=0, keepdims=True)
    nax = ((ax - ma) * lax.rsqrt(va + 1e-5) * gm_ref[:, C:]
           + bt_ref[:, C:])
    h = jnp.dot(ngr, w1_ref[0:C, :], preferred_element_type=jnp.float32)
    h = h + jnp.dot(nax, w1_ref[C:, :], preferred_element_type=jnp.float32)
    h = jnp.maximum(h + b1_ref[...], 0.0)
    out_ref[...] = (
        jnp.dot(h, w2_ref[...], preferred_element_type=jnp.float32)
        + b2_ref[...])


def kernel(initial_node_states, final_node_states, aux_variables, num_graphs,
           graph_nodes_list, Wg, bg, Wt, bt, gamma, beta, W1, b1, W2, b2):
    f32 = jnp.float32
    del num_graphs
    ids = jnp.asarray(graph_nodes_list, jnp.int32)

    nodewise = pl.pallas_call(
        _nodewise_body,
        grid=(NP // TCB,),
        in_specs=[
            pl.BlockSpec((TCB, H), lambda i: (i, 0)),
            pl.BlockSpec((TCB, H), lambda i: (i, 0)),
            pl.BlockSpec((2 * H, C), lambda i: (0, 0)),
            pl.BlockSpec((H, C), lambda i: (0, 0)),
            pl.BlockSpec((1, C), lambda i: (0, 0)),
            pl.BlockSpec((1, C), lambda i: (0, 0)),
        ],
        out_specs=pl.BlockSpec((MT, 128), lambda i: (i, 0)),
        out_shape=jax.ShapeDtypeStruct((NP // 8, 128), f32),
    )(initial_node_states, final_node_states, Wg,
      Wt, bg.reshape(1, C), bt.reshape(1, C))

    mesh = plsc.VectorSubcoreMesh(core_axis_name="c", subcore_axis_name="s")
    partials = pl.kernel(
        _segsum_body,
        out_type=jax.ShapeDtypeStruct((NW, NG // 8, 128), f32),
        mesh=mesh,
        scratch_types=[
            pltpu.VMEM((RPW, 128), f32),
            pltpu.VMEM((CHUNK + 2432, ), jnp.int32),
            pltpu.VMEM((NG // 8, 128), f32),
            pltpu.SemaphoreType.DMA,
            pltpu.SemaphoreType.DMA,
        ],
        compiler_params=pltpu.CompilerParams(
            needs_layout_passes=False, use_tc_tiling_on_sc=False),
    )(nodewise, ids)

    logits = pl.pallas_call(
        _finalize_body,
        out_shape=jax.ShapeDtypeStruct((NG, C), f32),
    )(partials, aux_variables, gamma.reshape(1, C + 2),
      beta.reshape(1, C + 2), W1, b1.reshape(1, GX), W2, b2.reshape(1, C))
    return logits
